# Initial kernel scaffold; baseline (speedup 1.0000x reference)
import jax
import jax.numpy as jnp
from jax.experimental import pallas as pl


def kernel(x, weights_embed, bias):
    raise NotImplementedError("write your pallas kernel here")


# speedup vs baseline: 1.2963x; 1.2963x over previous
import jax
import jax.numpy as jnp
from jax import lax
from jax.experimental import pallas as pl
from jax.experimental.pallas import tpu as pltpu
from jax.experimental.pallas import tpu_sc as plsc

NUM_CORES = 2
NUM_SUBCORES = 16
LANES = 16

B = 16384
F = 26
V = 100000
BH = B // NUM_CORES
ROWS = BH // 128
OROWS = ROWS // 8


def _lookup_body(xt_hbm, tab_hbm, bias_hbm, out_hbm,
                 sub_v, idx_v, part_v, iota_v, bias_v, outb_v, accum):
    c = lax.axis_index("c")
    s = lax.axis_index("s")

    for i in range(ROWS // LANES):
        iota_v[pl.ds(i * LANES, LANES)] = (
            lax.iota(jnp.int32, LANES) + i * LANES)

    @pl.when(s == 0)
    def _():
        def zero_row(r, _):
            for l in range(128 // LANES):
                part_v[r, pl.ds(l * LANES, LANES)] = (
                    jnp.zeros((LANES,), jnp.float32))
            return 0
        lax.fori_loop(0, ROWS, zero_row, 0)
        pltpu.sync_copy(part_v, accum)

    pltpu.sync_copy(bias_hbm, bias_v)
    plsc.subcore_barrier()

    def do_feature(f):
        pltpu.sync_copy(tab_hbm.at[pl.ds(pl.multiple_of(f * V, 8), V)], sub_v)
        pltpu.sync_copy(xt_hbm.at[f, pl.ds(pl.multiple_of(c * BH, 8), BH)],
                        idx_v)

        def gather_row(r, _):
            for l in range(128 // LANES):
                iv = idx_v[pl.ds(r * 128 + l * LANES, LANES)]
                part_v[r, pl.ds(l * LANES, LANES)] = (
                    plsc.load_gather(sub_v, [iv]))
            return 0
        lax.fori_loop(0, ROWS, gather_row, 0)

        pltpu.sync_copy(part_v, accum.at[iota_v], add=True)

    do_feature(s)

    @pl.when(s < F - NUM_SUBCORES)
    def _():
        do_feature(s + NUM_SUBCORES)

    plsc.subcore_barrier()

    @pl.when(s < ROWS // OROWS)
    def _():
        pltpu.sync_copy(
            accum.at[pl.ds(pl.multiple_of(s * OROWS, 8), OROWS), :], outb_v)
        bvec = bias_v[...]

        def add_bias(r, _):
            for l in range(128 // LANES):
                outb_v[r, pl.ds(l * LANES, LANES)] = (
                    outb_v[r, pl.ds(l * LANES, LANES)] + bvec)
            return 0
        lax.fori_loop(0, OROWS, add_bias, 0)

        row0 = pl.multiple_of(c * ROWS + s * OROWS, 8)
        pltpu.sync_copy(outb_v, out_hbm.at[pl.ds(row0, OROWS), :])


@jax.jit
def _run(xt, tab, bias16):
    mesh = plsc.VectorSubcoreMesh(
        core_axis_name="c", subcore_axis_name="s",
        num_cores=NUM_CORES, num_subcores=NUM_SUBCORES)
    return pl.kernel(
        _lookup_body,
        out_type=jax.ShapeDtypeStruct((B // 128, 128), jnp.float32),
        mesh=mesh,
        compiler_params=pltpu.CompilerParams(needs_layout_passes=False),
        scratch_types=[
            pltpu.VMEM((V,), jnp.float32),
            pltpu.VMEM((BH,), jnp.int32),
            pltpu.VMEM((ROWS, 128), jnp.float32),
            pltpu.VMEM((ROWS,), jnp.int32),
            pltpu.VMEM((LANES,), jnp.float32),
            pltpu.VMEM((OROWS, 128), jnp.float32),
            pltpu.VMEM_SHARED((ROWS, 128), jnp.float32),
        ],
    )(xt, tab, bias16)


def kernel(x, weights_embed, bias):
    xt = x.T
    tab = weights_embed.reshape(-1)
    bias16 = jnp.broadcast_to(bias, (LANES,))
    out = _run(xt, tab, bias16)
    return out.reshape(B, 1)
